# baseline (speedup 1.0000x reference)
import jax
import jax.numpy as jnp
from jax import lax
from jax.experimental import pallas as pl
from jax.experimental.pallas import tpu as pltpu
from jax.experimental.pallas import tpu_sc as plsc

_NC = 2
_NS = 16
_NW = _NC * _NS

_CHUNK = 128


def _build(num_rows: int, emb_dim: int):
  assert num_rows % (_NW * _CHUNK) == 0
  rows_per_w = num_rows // _NW
  n_chunks = rows_per_w // _CHUNK
  mesh = plsc.VectorSubcoreMesh(core_axis_name="c", subcore_axis_name="s")

  def body(idx_hbm, table_hbm, out_hbm, idx_v, buf0, buf1, sem0, sem1):
    wid = lax.axis_index("s") * _NC + lax.axis_index("c")
    base = wid * rows_per_w
    pltpu.sync_copy(idx_hbm.at[wid], idx_v)

    bufs = (buf0, buf1)
    sems = (sem0, sem1)

    def start(chunk, b):
      pltpu.async_copy(table_hbm.at[idx_v.at[chunk]], bufs[b], sems[b])

    def finish(chunk, b):
      pltpu.make_async_copy(table_hbm.at[idx_v.at[chunk]], bufs[b], sems[b]).wait()

    start(0, 0)
    start(1, 1)

    @pl.loop(0, n_chunks // 2 - 1)
    def _(g0):
      for b in range(2):
        g = g0 * 2 + b
        finish(g, b)
        pltpu.sync_copy(bufs[b], out_hbm.at[pl.ds(base + g * _CHUNK, _CHUNK)])
        start(g + 2, b)

    for b in range(2):
      g = n_chunks - 2 + b
      finish(g, b)
      pltpu.sync_copy(bufs[b], out_hbm.at[pl.ds(base + g * _CHUNK, _CHUNK)])

  return pl.kernel(
      body,
      out_type=jax.ShapeDtypeStruct((num_rows, emb_dim), jnp.float32),
      mesh=mesh,
      scratch_types=[
          pltpu.VMEM((n_chunks, _CHUNK), jnp.int32),
          pltpu.VMEM((_CHUNK, emb_dim), jnp.float32),
          pltpu.VMEM((_CHUNK, emb_dim), jnp.float32),
          pltpu.SemaphoreType.DMA,
          pltpu.SemaphoreType.DMA,
      ],
      compiler_params=pltpu.CompilerParams(use_tc_tiling_on_sc=False),
  )


def kernel(indices, table):
  batch, seq = indices.shape
  vocab, emb_dim = table.shape
  num_rows = batch * seq
  idx3 = indices.astype(jnp.int32).reshape(_NW, num_rows // (_NW * _CHUNK), _CHUNK)
  out = _build(num_rows, emb_dim)(idx3, table)
  return out.reshape(batch, seq, emb_dim)
